# baseline (speedup 1.0000x reference)
import functools

import jax
import jax.numpy as jnp
from jax import lax
from jax.experimental import pallas as pl
from jax.experimental.pallas import tpu as pltpu
from jax.experimental.pallas import tpu_sc as plsc

NUM_ROWS = 1000000
DIM = 64
BATCH = 16384

_NC = 2
_NS = 16
_NW = _NC * _NS
_B_PER_W = BATCH // _NW
_IDX_CHUNK = 128
_N_CHUNKS = _B_PER_W // _IDX_CHUNK

_SUM_BLK = 10000
_FIN_BLK = 2048


def _sc_gather_body(table_hbm, idx_hbm, out_hbm, idx_v, rows_v, sem):
    wid = lax.axis_index("s") * _NC + lax.axis_index("c")
    base = wid * _B_PER_W
    pltpu.sync_copy(idx_hbm.at[pl.ds(base, _B_PER_W)], idx_v)
    copies = [
        pltpu.async_copy(
            table_hbm.at[idx_v.at[pl.ds(j * _IDX_CHUNK, _IDX_CHUNK)]],
            rows_v.at[pl.ds(j * _IDX_CHUNK, _IDX_CHUNK)],
            sem,
        )
        for j in range(_N_CHUNKS)
    ]
    for c in copies:
        c.wait()
    pltpu.sync_copy(rows_v, out_hbm.at[pl.ds(base, _B_PER_W)])


_sc_gather = pl.kernel(
    _sc_gather_body,
    mesh=plsc.VectorSubcoreMesh(core_axis_name="c", subcore_axis_name="s"),
    compiler_params=pltpu.CompilerParams(use_tc_tiling_on_sc=False),
    out_type=jax.ShapeDtypeStruct((BATCH, DIM), jnp.float32),
    scratch_types=[
        pltpu.VMEM((_B_PER_W,), jnp.int32),
        pltpu.VMEM((_B_PER_W, DIM), jnp.float32),
        pltpu.SemaphoreType.DMA,
    ],
)


def _colsum_body(x_ref, o_ref):
    @pl.when(pl.program_id(0) == 0)
    def _init():
        o_ref[...] = jnp.zeros_like(o_ref)

    x = x_ref[...]
    o_ref[...] += jnp.sum(x.reshape(_SUM_BLK // 8, 8, 128), axis=0)


def _colsum(table2):
    n_blocks = table2.shape[0] // _SUM_BLK
    return pl.pallas_call(
        _colsum_body,
        grid=(n_blocks,),
        in_specs=[pl.BlockSpec((_SUM_BLK, 128), lambda i: (i, 0))],
        out_specs=pl.BlockSpec((8, 128), lambda i: (0, 0)),
        out_shape=jax.ShapeDtypeStruct((8, 128), jnp.float32),
    )(table2)


def _finalize_body(raw_ref, mean_ref, o_ref):
    x = raw_ref[...] - mean_ref[0:1, :]
    n2 = jnp.sum(x * x, axis=1, keepdims=True)
    o_ref[...] = x * lax.rsqrt(jnp.maximum(n2, 1e-24))


def _finalize(raw, mean_b):
    return pl.pallas_call(
        _finalize_body,
        grid=(BATCH // _FIN_BLK,),
        in_specs=[
            pl.BlockSpec((_FIN_BLK, DIM), lambda i: (i, 0)),
            pl.BlockSpec((8, DIM), lambda i: (0, 0)),
        ],
        out_specs=pl.BlockSpec((_FIN_BLK, DIM), lambda i: (i, 0)),
        out_shape=jax.ShapeDtypeStruct((BATCH, DIM), jnp.float32),
    )(raw, mean_b)


def kernel(indices, embedding):
    idx = indices.astype(jnp.int32)
    raw = _sc_gather(embedding, idx)
    table2 = embedding.reshape(NUM_ROWS // 2, 2 * DIM)
    s = jnp.sum(_colsum(table2), axis=0)
    mean64 = (s[:DIM] + s[DIM:]) * (1.0 / NUM_ROWS)
    mean_b = jnp.broadcast_to(mean64[None, :], (8, DIM))
    return _finalize(raw, mean_b)


# speedup vs baseline: 1.3719x; 1.3719x over previous
import functools

import jax
import jax.numpy as jnp
from jax import lax
from jax.experimental import pallas as pl
from jax.experimental.pallas import tpu as pltpu
from jax.experimental.pallas import tpu_sc as plsc

NUM_ROWS = 1000000
DIM = 64
BATCH = 16384

_NC = 2
_NS = 16
_NW = _NC * _NS
_B_PER_W = BATCH // _NW
_IDX_CHUNK = 128
_N_CHUNKS = _B_PER_W // _IDX_CHUNK

_SUM_BLK = 4096
_SUM_GRID = (NUM_ROWS + _SUM_BLK - 1) // _SUM_BLK
_SUM_REM = NUM_ROWS - (_SUM_GRID - 1) * _SUM_BLK
_FIN_BLK = 2048


def _sc_gather_body(table_hbm, idx_hbm, out_hbm, idx_v, rows_v, sem):
    wid = lax.axis_index("s") * _NC + lax.axis_index("c")
    base = wid * _B_PER_W
    pltpu.sync_copy(idx_hbm.at[pl.ds(base, _B_PER_W)], idx_v)
    copies = [
        pltpu.async_copy(
            table_hbm.at[idx_v.at[pl.ds(j * _IDX_CHUNK, _IDX_CHUNK)]],
            rows_v.at[pl.ds(j * _IDX_CHUNK, _IDX_CHUNK)],
            sem,
        )
        for j in range(_N_CHUNKS)
    ]
    for c in copies:
        c.wait()
    pltpu.sync_copy(rows_v, out_hbm.at[pl.ds(base, _B_PER_W)])


_sc_gather = pl.kernel(
    _sc_gather_body,
    mesh=plsc.VectorSubcoreMesh(core_axis_name="c", subcore_axis_name="s"),
    compiler_params=pltpu.CompilerParams(use_tc_tiling_on_sc=False),
    out_type=jax.ShapeDtypeStruct((BATCH, DIM), jnp.float32),
    scratch_types=[
        pltpu.VMEM((_B_PER_W,), jnp.int32),
        pltpu.VMEM((_B_PER_W, DIM), jnp.float32),
        pltpu.SemaphoreType.DMA,
    ],
)


def _colsum_body(x_ref, o_ref):
    j = pl.program_id(0)

    @pl.when(j == 0)
    def _init():
        o_ref[...] = jnp.zeros_like(o_ref)

    x = x_ref[...]

    @pl.when(j < _SUM_GRID - 1)
    def _full():
        s = x[:, 0:128]
        for k in range(1, _SUM_BLK // 128):
            s = s + x[:, k * 128:(k + 1) * 128]
        o_ref[...] += s

    @pl.when(j == _SUM_GRID - 1)
    def _tail():
        n_full = _SUM_REM // 128
        s = x[:, 0:128]
        for k in range(1, n_full):
            s = s + x[:, k * 128:(k + 1) * 128]
        part = _SUM_REM - n_full * 128
        if part:
            tail = x[:, n_full * 128:(n_full + 1) * 128]
            lane = lax.broadcasted_iota(jnp.int32, (DIM, 128), 1)
            s = s + jnp.where(lane < part, tail, 0.0)
        o_ref[...] += s


def _colsum(table_t):
    return pl.pallas_call(
        _colsum_body,
        grid=(_SUM_GRID,),
        in_specs=[pl.BlockSpec((DIM, _SUM_BLK), lambda i: (0, i))],
        out_specs=pl.BlockSpec((DIM, 128), lambda i: (0, 0)),
        out_shape=jax.ShapeDtypeStruct((DIM, 128), jnp.float32),
    )(table_t)


def _finalize_body(raw_ref, mean_ref, o_ref):
    x = raw_ref[...] - mean_ref[0:1, :]
    n2 = jnp.sum(x * x, axis=1, keepdims=True)
    o_ref[...] = x * lax.rsqrt(jnp.maximum(n2, 1e-24))


def _finalize(raw, mean_b):
    return pl.pallas_call(
        _finalize_body,
        grid=(BATCH // _FIN_BLK,),
        in_specs=[
            pl.BlockSpec((_FIN_BLK, DIM), lambda i: (i, 0)),
            pl.BlockSpec((8, DIM), lambda i: (0, 0)),
        ],
        out_specs=pl.BlockSpec((_FIN_BLK, DIM), lambda i: (i, 0)),
        out_shape=jax.ShapeDtypeStruct((BATCH, DIM), jnp.float32),
    )(raw, mean_b)


def kernel(indices, embedding):
    idx = indices.astype(jnp.int32)
    raw = _sc_gather(embedding, idx)
    acc = _colsum(embedding.T)
    mean64 = jnp.sum(acc, axis=1) * (1.0 / NUM_ROWS)
    mean_b = jnp.broadcast_to(mean64[None, :], (8, DIM))
    return _finalize(raw, mean_b)


# speedup vs baseline: 1.5012x; 1.0943x over previous
import functools

import jax
import jax.numpy as jnp
from jax import lax
from jax.experimental import pallas as pl
from jax.experimental.pallas import tpu as pltpu
from jax.experimental.pallas import tpu_sc as plsc

NUM_ROWS = 1000000
DIM = 64
BATCH = 16384

_NC = 2
_NS = 16
_NW = _NC * _NS
_B_PER_W = BATCH // _NW
_IDX_CHUNK = 128
_N_CHUNKS = _B_PER_W // _IDX_CHUNK

_SUM_BLK = 4096
_SUM_GRID = (NUM_ROWS + _SUM_BLK - 1) // _SUM_BLK
_SUM_REM = NUM_ROWS - (_SUM_GRID - 1) * _SUM_BLK
_FIN_BLK = 2048


def _sc_gather_body(table_hbm, idx_hbm, out_hbm, idx_v, rows_v, sem):
    wid = lax.axis_index("s") * _NC + lax.axis_index("c")
    base = wid * _B_PER_W
    pltpu.sync_copy(idx_hbm.at[pl.ds(base, _B_PER_W)], idx_v)
    copies = [
        pltpu.async_copy(
            table_hbm.at[idx_v.at[pl.ds(j * _IDX_CHUNK, _IDX_CHUNK)]],
            rows_v.at[pl.ds(j * _IDX_CHUNK, _IDX_CHUNK)],
            sem,
        )
        for j in range(_N_CHUNKS)
    ]
    for c in copies:
        c.wait()
    pltpu.sync_copy(rows_v, out_hbm.at[pl.ds(base, _B_PER_W)])


_sc_gather = pl.kernel(
    _sc_gather_body,
    mesh=plsc.VectorSubcoreMesh(core_axis_name="c", subcore_axis_name="s"),
    compiler_params=pltpu.CompilerParams(use_tc_tiling_on_sc=True),
    out_type=jax.ShapeDtypeStruct((BATCH, 2 * DIM), jnp.float32),
    scratch_types=[
        pltpu.VMEM((_B_PER_W,), jnp.int32),
        pltpu.VMEM((_B_PER_W, 2 * DIM), jnp.float32),
        pltpu.SemaphoreType.DMA,
    ],
)


def _colsum_body(x_ref, o_ref):
    j = pl.program_id(0)

    @pl.when(j == 0)
    def _init():
        o_ref[...] = jnp.zeros_like(o_ref)

    x = x_ref[...]

    @pl.when(j < _SUM_GRID - 1)
    def _full():
        s = x[:, 0:128]
        for k in range(1, _SUM_BLK // 128):
            s = s + x[:, k * 128:(k + 1) * 128]
        o_ref[...] += s

    @pl.when(j == _SUM_GRID - 1)
    def _tail():
        n_full = _SUM_REM // 128
        s = x[:, 0:128]
        for k in range(1, n_full):
            s = s + x[:, k * 128:(k + 1) * 128]
        part = _SUM_REM - n_full * 128
        if part:
            tail = x[:, n_full * 128:(n_full + 1) * 128]
            lane = lax.broadcasted_iota(jnp.int32, (DIM, 128), 1)
            s = s + jnp.where(lane < part, tail, 0.0)
        o_ref[...] += s


def _colsum(table_t):
    return pl.pallas_call(
        _colsum_body,
        grid=(_SUM_GRID,),
        in_specs=[pl.BlockSpec((DIM, _SUM_BLK), lambda i: (0, i))],
        out_specs=pl.BlockSpec((DIM, 128), lambda i: (0, 0)),
        out_shape=jax.ShapeDtypeStruct((DIM, 128), jnp.float32),
    )(table_t)


def _finalize_body(raw_ref, mean_ref, o_ref):
    x = raw_ref[:, :DIM] - mean_ref[0:1, :]
    n2 = jnp.sum(x * x, axis=1, keepdims=True)
    o_ref[...] = x * lax.rsqrt(jnp.maximum(n2, 1e-24))


def _finalize(raw, mean_b):
    return pl.pallas_call(
        _finalize_body,
        grid=(BATCH // _FIN_BLK,),
        in_specs=[
            pl.BlockSpec((_FIN_BLK, 2 * DIM), lambda i: (i, 0)),
            pl.BlockSpec((8, DIM), lambda i: (0, 0)),
        ],
        out_specs=pl.BlockSpec((_FIN_BLK, DIM), lambda i: (i, 0)),
        out_shape=jax.ShapeDtypeStruct((BATCH, DIM), jnp.float32),
    )(raw, mean_b)


def kernel(indices, embedding):
    idx = indices.astype(jnp.int32)
    table2 = jnp.pad(embedding, ((0, 0), (0, DIM)))
    raw2 = _sc_gather(table2, idx)
    acc = _colsum(embedding.T)
    mean64 = jnp.sum(acc, axis=1) * (1.0 / NUM_ROWS)
    mean_b = jnp.broadcast_to(mean64[None, :], (8, DIM))
    return _finalize(raw2, mean_b)


# speedup vs baseline: 2.9588x; 1.9709x over previous
import functools

import jax
import jax.numpy as jnp
from jax import lax
from jax.experimental import pallas as pl
from jax.experimental.pallas import tpu as pltpu
from jax.experimental.pallas import tpu_sc as plsc

NUM_ROWS = 1000000
DIM = 64
BATCH = 16384

_NC = 2
_NS = 16
_NW = _NC * _NS
_B_PER_W = BATCH // _NW
_IDX_CHUNK = 128
_N_CHUNKS = _B_PER_W // _IDX_CHUNK

_SUM_BLK = 4096
_SUM_GRID = (NUM_ROWS + _SUM_BLK - 1) // _SUM_BLK
_SUM_REM = NUM_ROWS - (_SUM_GRID - 1) * _SUM_BLK
_FIN_BLK = 2048


def _sc_gather_body(table_hbm, idx_hbm, out_hbm, idx_v, rows_v, sem):
    wid = lax.axis_index("s") * _NC + lax.axis_index("c")
    base = wid * _B_PER_W
    pltpu.sync_copy(idx_hbm.at[pl.ds(base, _B_PER_W)], idx_v)
    copies = [
        pltpu.async_copy(
            table_hbm.at[idx_v.at[pl.ds(j * _IDX_CHUNK, _IDX_CHUNK)]],
            rows_v.at[pl.ds(j * _IDX_CHUNK, _IDX_CHUNK)],
            sem,
        )
        for j in range(_N_CHUNKS)
    ]
    for c in copies:
        c.wait()
    pltpu.sync_copy(rows_v, out_hbm.at[pl.ds(base, _B_PER_W)])


_sc_gather = pl.kernel(
    _sc_gather_body,
    mesh=plsc.VectorSubcoreMesh(core_axis_name="c", subcore_axis_name="s"),
    compiler_params=pltpu.CompilerParams(use_tc_tiling_on_sc=True),
    out_type=jax.ShapeDtypeStruct((BATCH, 2 * DIM), jnp.float32),
    scratch_types=[
        pltpu.VMEM((_B_PER_W,), jnp.int32),
        pltpu.VMEM((_B_PER_W, 2 * DIM), jnp.float32),
        pltpu.SemaphoreType.DMA,
    ],
)


def _prep_body(x_ref, y_ref, o_ref):
    j = pl.program_id(0)

    @pl.when(j == 0)
    def _init():
        o_ref[...] = jnp.zeros_like(o_ref)

    x = x_ref[...]
    y_ref[...] = jnp.concatenate([x, x], axis=0).T

    @pl.when(j < _SUM_GRID - 1)
    def _full():
        s = x[:, 0:128]
        for k in range(1, _SUM_BLK // 128):
            s = s + x[:, k * 128:(k + 1) * 128]
        o_ref[...] += s

    @pl.when(j == _SUM_GRID - 1)
    def _tail():
        n_full = _SUM_REM // 128
        s = x[:, 0:128]
        for k in range(1, n_full):
            s = s + x[:, k * 128:(k + 1) * 128]
        part = _SUM_REM - n_full * 128
        if part:
            tail = x[:, n_full * 128:(n_full + 1) * 128]
            lane = lax.broadcasted_iota(jnp.int32, (DIM, 128), 1)
            s = s + jnp.where(lane < part, tail, 0.0)
        o_ref[...] += s


def _prep(table_t):
    return pl.pallas_call(
        _prep_body,
        grid=(_SUM_GRID,),
        in_specs=[pl.BlockSpec((DIM, _SUM_BLK), lambda i: (0, i))],
        out_specs=[
            pl.BlockSpec((_SUM_BLK, 2 * DIM), lambda i: (i, 0)),
            pl.BlockSpec((DIM, 128), lambda i: (0, 0)),
        ],
        out_shape=[
            jax.ShapeDtypeStruct((NUM_ROWS, 2 * DIM), jnp.float32),
            jax.ShapeDtypeStruct((DIM, 128), jnp.float32),
        ],
    )(table_t)


def _finalize_body(raw_ref, mean_ref, o_ref):
    x = raw_ref[:, :DIM] - mean_ref[0:1, :]
    n2 = jnp.sum(x * x, axis=1, keepdims=True)
    o_ref[...] = x * lax.rsqrt(jnp.maximum(n2, 1e-24))


def _finalize(raw, mean_b):
    return pl.pallas_call(
        _finalize_body,
        grid=(BATCH // _FIN_BLK,),
        in_specs=[
            pl.BlockSpec((_FIN_BLK, 2 * DIM), lambda i: (i, 0)),
            pl.BlockSpec((8, DIM), lambda i: (0, 0)),
        ],
        out_specs=pl.BlockSpec((_FIN_BLK, DIM), lambda i: (i, 0)),
        out_shape=jax.ShapeDtypeStruct((BATCH, DIM), jnp.float32),
    )(raw, mean_b)


def kernel(indices, embedding):
    idx = indices.astype(jnp.int32)
    table2, acc = _prep(embedding.T)
    raw2 = _sc_gather(table2, idx)
    mean64 = jnp.sum(acc, axis=1) * (1.0 / NUM_ROWS)
    mean_b = jnp.broadcast_to(mean64[None, :], (8, DIM))
    return _finalize(raw2, mean_b)
